# baseline (speedup 1.0000x reference)
import functools

import jax
import jax.numpy as jnp
from jax import lax
from jax.experimental import pallas as pl
from jax.experimental.pallas import tpu as pltpu
from jax.experimental.pallas import tpu_sc as plsc

_N = 10000
_E = 320000
_D = 128

_NC = 2
_NS = 16
_CH = 64
_CPT = 162
_PKR = _CPT // 2
_PKA = 88
_EPT = _CH * _CPT
_NT = _NC * _NS
_E_PAD = _EPT * _NT
_NPAD = 10240
_RP = _NPAD // _NS
_SHIFT = 14
_MASK = (1 << _SHIFT) - 1


def _mm_body(x_ref, w_ref, o_ref):
    o_ref[...] = jnp.dot(x_ref[...], w_ref[...],
                         preferred_element_type=jnp.float32)


def _matmul(x, W):
    return pl.pallas_call(
        _mm_body,
        grid=(10,),
        in_specs=[
            pl.BlockSpec((1000, _D), lambda r: (r, 0)),
            pl.BlockSpec((_D, _D), lambda r: (0, 0)),
        ],
        out_specs=pl.BlockSpec((1000, _D), lambda r: (r, 0)),
        out_shape=jax.ShapeDtypeStruct((_N, _D), jnp.float32),
    )(x, W)


def _add_body(a_ref, b_ref, o_ref):
    o_ref[...] = a_ref[0] + b_ref[0]


def _combine(parts):
    return pl.pallas_call(
        _add_body,
        grid=(10,),
        in_specs=[
            pl.BlockSpec((1, 1000, _D), lambda r: (0, r, 0)),
            pl.BlockSpec((1, 1000, _D), lambda r: (1, r, 0)),
        ],
        out_specs=pl.BlockSpec((1000, _D), lambda r: (r, 0)),
        out_shape=jax.ShapeDtypeStruct((_N, _D), jnp.float32),
    )(parts, parts)


@functools.partial(
    pl.kernel,
    out_type=jax.ShapeDtypeStruct((_NC, _NPAD, _D), jnp.float32),
    mesh=plsc.VectorSubcoreMesh(core_axis_name="c", subcore_axis_name="s"),
    scratch_types=[
        pltpu.VMEM((_PKA, 2 * _CH), jnp.int32),
        pltpu.VMEM((_PKA, 2 * _CH), jnp.float32),
        pltpu.VMEM((6, _CH), jnp.int32),
        pltpu.VMEM((6, _CH), jnp.int32),
        pltpu.VMEM((_CH, _D), jnp.float32),
        pltpu.VMEM((_CH, _D), jnp.float32),
        pltpu.VMEM((_CH, _D), jnp.float32),
        pltpu.VMEM_SHARED((_NPAD, _D), jnp.float32),
        pltpu.SemaphoreType.DMA,
        pltpu.SemaphoreType.DMA,
        pltpu.SemaphoreType.DMA,
        pltpu.SemaphoreType.DMA,
        pltpu.SemaphoreType.DMA,
        pltpu.SemaphoreType.DMA,
    ],
)
def _sc_agg(h_hbm, pk_hbm, vals_hbm, out_hbm,
            pk, val, col_st, row_st, rb0, rb1, rb2, acc,
            gs0, gs1, gs2, ss0, ss1, ss2):
    c = lax.axis_index("c")
    s = lax.axis_index("s")
    rbufs = (rb0, rb1, rb2)
    gsems = (gs0, gs1, gs2)
    ssems = (ss0, ss1, ss2)
    tid = c * _NS + s

    pltpu.sync_copy(pk_hbm.at[tid], pk)
    pltpu.sync_copy(vals_hbm.at[tid], val)

    def unpack(q, slot):
        r = q // 2
        base = (q % 2) * _CH
        for j in range(_CH // 16):
            pv = pk[r, pl.ds(base + j * 16, 16)]
            col_st[slot, pl.ds(j * 16, 16)] = jnp.bitwise_and(pv, _MASK)
            row_st[slot, pl.ds(j * 16, 16)] = jnp.right_shift(pv, _SHIFT)

    def issue_gather(b, slot):
        pltpu.async_copy(h_hbm.at[col_st.at[slot]], rbufs[b], gsems[b])

    def wait_gather(b):
        pltpu.make_async_copy(h_hbm.at[col_st.at[0]], rbufs[b],
                              gsems[b]).wait()

    def issue_scatter(b, slot):
        pltpu.async_copy(rbufs[b], acc.at[row_st.at[slot]], ssems[b],
                         add=True)

    def wait_scatter(b):
        pltpu.make_async_copy(rbufs[0], acc.at[row_st.at[0]],
                              ssems[b]).wait()

    def compute(g, t):
        rb = rbufs[t]
        r = g // 2
        base = (g % 2) * _CH

        def _edge16(g16, carry):
            vv = val[r, pl.ds(base + g16 * 16, 16)]
            for i in range(16):
                e = g16 * 16 + i
                sp = vv[i]
                for j in range(_D // 16):
                    rb[e, pl.ds(j * 16, 16)] = rb[e, pl.ds(j * 16, 16)] * sp
            return carry
        lax.fori_loop(0, _CH // 16, _edge16, 0)

    def _zrow(r, carry):
        for j in range(_D // 16):
            rb2[r, pl.ds(j * 16, 16)] = jnp.zeros((16,), jnp.float32)
        return carry
    lax.fori_loop(0, _CH, _zrow, 0)
    for k in range(_RP // _CH):
        pltpu.sync_copy(rb2, acc.at[pl.ds(s * _RP + k * _CH, _CH), :])
    for j in range(_CH // 16):
        row_st[3, pl.ds(j * 16, 16)] = jnp.zeros((16,), jnp.int32)

    unpack(0, 0)
    unpack(1, 1)
    unpack(2, 2)
    issue_gather(0, 0)
    issue_gather(1, 1)
    plsc.subcore_barrier()
    issue_scatter(2, 3)

    def _six(m, carry):
        for t6 in range(6):
            b = t6 % 3
            g = m * 6 + t6
            wait_gather(b)
            compute(g, b)
            issue_scatter(b, t6)
            wait_scatter((b + 2) % 3)
            unpack(g + 3, (t6 + 3) % 6)
            issue_gather((b + 2) % 3, (t6 + 2) % 6)
        return carry
    lax.fori_loop(0, _CPT // 6, _six, 0)
    wait_scatter(2)
    wait_gather(0)
    wait_gather(1)

    plsc.subcore_barrier()
    pltpu.sync_copy(acc.at[pl.ds(s * _RP, _RP), :],
                    out_hbm.at[c, pl.ds(s * _RP, _RP), :])


def kernel(x, edge_index, adj_values, W):
    ei = edge_index.astype(jnp.int32)
    pad = _E_PAD - _E
    packed = jnp.left_shift(ei[0], _SHIFT) | ei[1]
    pk_p = jnp.pad(packed, (0, pad)).reshape(_NT, _PKR, 2 * _CH)
    pk_p = jnp.pad(pk_p, ((0, 0), (0, _PKA - _PKR), (0, 0)))
    vals_p = jnp.pad(adj_values, (0, pad)).reshape(_NT, _PKR, 2 * _CH)
    vals_p = jnp.pad(vals_p, ((0, 0), (0, _PKA - _PKR), (0, 0)))
    h = _matmul(x, W)
    parts = _sc_agg(h, pk_p, vals_p)
    return _combine(parts)


# speedup vs baseline: 2.4596x; 2.4596x over previous
import functools

import jax
import jax.numpy as jnp
from jax import lax
from jax.experimental import pallas as pl
from jax.experimental.pallas import tpu as pltpu
from jax.experimental.pallas import tpu_sc as plsc

_N = 10000
_E = 320000
_D = 128

_NC = 2
_NS = 16
_CH = 112
_CPT = 90
_EPT = _CH * _CPT
_NT = _NC * _NS
_E_PAD = _EPT * _NT
_SLACK = 2 * _CH
_NPAD = 10240
_RP = _NPAD // _NS


def _mm_body(x_ref, w_ref, o_ref):
    o_ref[...] = jnp.dot(x_ref[...], w_ref[...],
                         preferred_element_type=jnp.float32)


def _matmul(x, W):
    return pl.pallas_call(
        _mm_body,
        grid=(10,),
        in_specs=[
            pl.BlockSpec((1000, _D), lambda r: (r, 0)),
            pl.BlockSpec((_D, _D), lambda r: (0, 0)),
        ],
        out_specs=pl.BlockSpec((1000, _D), lambda r: (r, 0)),
        out_shape=jax.ShapeDtypeStruct((_N, _D), jnp.float32),
    )(x, W)


def _add_body(a_ref, b_ref, o_ref):
    o_ref[...] = a_ref[0] + b_ref[0]


def _combine(parts):
    return pl.pallas_call(
        _add_body,
        grid=(10,),
        in_specs=[
            pl.BlockSpec((1, 1000, _D), lambda r: (0, r, 0)),
            pl.BlockSpec((1, 1000, _D), lambda r: (1, r, 0)),
        ],
        out_specs=pl.BlockSpec((1000, _D), lambda r: (r, 0)),
        out_shape=jax.ShapeDtypeStruct((_N, _D), jnp.float32),
    )(parts, parts)


@functools.partial(
    pl.kernel,
    out_type=jax.ShapeDtypeStruct((_NC, _NPAD, _D), jnp.float32),
    mesh=plsc.VectorSubcoreMesh(core_axis_name="c", subcore_axis_name="s"),
    scratch_types=[
        pltpu.VMEM((_CH,), jnp.int32),
        pltpu.VMEM((_CH,), jnp.int32),
        pltpu.VMEM((_CH,), jnp.int32),
        pltpu.VMEM((_CH,), jnp.int32),
        pltpu.VMEM((_CH,), jnp.int32),
        pltpu.VMEM((_CH,), jnp.int32),
        pltpu.VMEM((_CH,), jnp.float32),
        pltpu.VMEM((_CH,), jnp.float32),
        pltpu.VMEM((_CH,), jnp.float32),
        pltpu.VMEM((_CH, _D), jnp.float32),
        pltpu.VMEM((_CH, _D), jnp.float32),
        pltpu.VMEM((_CH, _D), jnp.float32),
        pltpu.VMEM_SHARED((_NPAD, _D), jnp.float32),
        pltpu.SemaphoreType.DMA,
        pltpu.SemaphoreType.DMA,
        pltpu.SemaphoreType.DMA,
        pltpu.SemaphoreType.DMA,
        pltpu.SemaphoreType.DMA,
        pltpu.SemaphoreType.DMA,
    ],
)
def _sc_agg(hs_hbm, cols_hbm, rows_hbm, vals_hbm, out_hbm,
            cb0, cb1, cb2, rw0, rw1, rw2, vb0, vb1, vb2,
            rb0, rb1, rb2, acc,
            gs0, gs1, gs2, ss0, ss1, ss2):
    c = lax.axis_index("c")
    s = lax.axis_index("s")
    cbufs = (cb0, cb1, cb2)
    rwbufs = (rw0, rw1, rw2)
    vbufs = (vb0, vb1, vb2)
    rbufs = (rb0, rb1, rb2)
    gsems = (gs0, gs1, gs2)
    ssems = (ss0, ss1, ss2)
    tile_base = (c * _NS + s) * _EPT

    def idx_dma(q, k):
        base = tile_base + q * _CH
        pltpu.sync_copy(cols_hbm.at[pl.ds(base, _CH)], cbufs[k])
        pltpu.sync_copy(rows_hbm.at[pl.ds(base, _CH)], rwbufs[k])
        pltpu.sync_copy(vals_hbm.at[pl.ds(base, _CH)], vbufs[k])

    def issue_gather(k):
        pltpu.async_copy(hs_hbm.at[cbufs[k]], rbufs[k], gsems[k])

    def wait_gather(k):
        pltpu.make_async_copy(hs_hbm.at[cbufs[0]], rbufs[k],
                              gsems[k]).wait()

    def issue_scatter(k):
        pltpu.async_copy(rbufs[k], acc.at[rwbufs[k]], ssems[k], add=True)

    def wait_scatter(k):
        pltpu.make_async_copy(rbufs[0], acc.at[rwbufs[0]],
                              ssems[k]).wait()

    def compute(k):
        rb = rbufs[k]
        vb = vbufs[k]

        def _edge16(g16, carry):
            vv = vb[pl.ds(g16 * 16, 16)]
            for i in range(16):
                e = g16 * 16 + i
                sp = vv[i]
                for j in range(_D // 16):
                    rb[e, pl.ds(j * 16, 16)] = rb[e, pl.ds(j * 16, 16)] * sp
            return carry
        lax.fori_loop(0, _CH // 16, _edge16, 0)

    def _zrow(r, carry):
        for j in range(_D // 16):
            rb2[r, pl.ds(j * 16, 16)] = jnp.zeros((16,), jnp.float32)
        return carry
    lax.fori_loop(0, _CH, _zrow, 0)
    for k in range(_RP // _CH):
        pltpu.sync_copy(rb2, acc.at[pl.ds(s * _RP + k * _CH, _CH), :])
    if _RP % _CH:
        pltpu.sync_copy(
            rb2.at[pl.ds(0, _RP % _CH), :],
            acc.at[pl.ds(s * _RP + (_RP // _CH) * _CH, _RP % _CH), :])
    for j in range(_CH // 16):
        rw2[pl.ds(j * 16, 16)] = jnp.zeros((16,), jnp.int32)

    idx_dma(0, 0)
    idx_dma(1, 1)
    issue_gather(0)
    issue_gather(1)
    plsc.subcore_barrier()
    issue_scatter(2)

    def _step(g, k):
        wait_gather(k)
        compute(k)
        issue_scatter(k)
        wait_scatter((k + 2) % 3)
        idx_dma(g + 2, (k + 2) % 3)
        issue_gather((k + 2) % 3)

    def _triple(m, carry):
        for k in range(3):
            _step(m * 3 + k, k)
        return carry
    lax.fori_loop(0, _CPT // 3, _triple, 0)
    wait_scatter((_CPT - 1) % 3)
    wait_gather(_CPT % 3)
    wait_gather((_CPT + 1) % 3)

    plsc.subcore_barrier()
    pltpu.sync_copy(acc.at[pl.ds(s * _RP, _RP), :],
                    out_hbm.at[c, pl.ds(s * _RP, _RP), :])


def kernel(x, edge_index, adj_values, W):
    ei = edge_index.astype(jnp.int32)
    pad = _E_PAD + _SLACK - _E
    rows_p = jnp.pad(ei[0], (0, pad))
    cols_p = jnp.pad(ei[1], (0, pad))
    vals_p = jnp.pad(adj_values, (0, pad))
    h = _matmul(x, W)
    parts = _sc_agg(h, cols_p, rows_p, vals_p)
    return _combine(parts)
